# Initial kernel scaffold; baseline (speedup 1.0000x reference)
import jax
import jax.numpy as jnp
from jax.experimental import pallas as pl


def kernel(node_pos, edge_index):
    raise NotImplementedError("write your pallas kernel here")


# speedup vs baseline: 1.2395x; 1.2395x over previous
import functools

import jax
import jax.numpy as jnp
from jax.experimental import pallas as pl
from jax.experimental.pallas import tpu as pltpu

_THRESH = 0.1
_BM = 2048
_BN = 2048
_L = 128


def _count_kernel(nj, xbc_ref, ybc_ref, xrow_ref, yrow_ref, out_ref,
                  xn_ref, yn_ref, acc_ref):
    bi = pl.program_id(0)
    bj = pl.program_id(1)

    @pl.when(bj == 0)
    def _init():
        dx = xbc_ref[...]
        dy = ybc_ref[...]
        norm = jnp.sqrt(dx * dx + dy * dy)
        inv = 1.0 / jnp.maximum(norm, 1e-6)
        xn_ref[...] = dx * inv
        yn_ref[...] = dy * inv
        acc_ref[...] = jnp.zeros_like(acc_ref)

    rx = xrow_ref[...]
    ry = yrow_ref[...]
    rinv = 1.0 / jnp.maximum(jnp.sqrt(rx * rx + ry * ry), 1e-6)
    rxn = rx * rinv
    ryn = ry * rinv

    xn = xn_ref[...]
    yn = yn_ref[...]

    for c in range(_BN // _L):
        xj = rxn[:, c * _L:(c + 1) * _L]
        yj = ryn[:, c * _L:(c + 1) * _L]
        t = xn * xj + yn * yj
        hf = jnp.where(jnp.abs(t) > _THRESH, 1.0, 0.0)
        acc_ref[...] += hf

        @pl.when(bi == bj)
        def _diag(c=c, hf=hf):
            sub = hf[c * _L:(c + 1) * _L, :]
            r = jax.lax.broadcasted_iota(jnp.int32, (_L, _L), 0)
            k = jax.lax.broadcasted_iota(jnp.int32, (_L, _L), 1)
            acc_ref[c * _L:(c + 1) * _L, :] -= jnp.where(r == k, sub, 0.0)

    @pl.when(bj == nj - 1)
    def _flush():
        a = acc_ref[...].reshape(_BM // 8, 8, _L)
        out_ref[...] = jnp.sum(a, axis=0).reshape(1, 8, _L)


@jax.jit
def kernel(node_pos, edge_index):
    e = edge_index.shape[1]
    d = node_pos[edge_index[1]] - node_pos[edge_index[0]]
    dx = d[:, 0]
    dy = d[:, 1]
    xbc = jnp.broadcast_to(dx[:, None], (e, _L))
    ybc = jnp.broadcast_to(dy[:, None], (e, _L))
    xrow = dx[None, :]
    yrow = dy[None, :]
    ni = e // _BM
    nj = e // _BN
    out = pl.pallas_call(
        functools.partial(_count_kernel, nj),
        grid=(ni, nj),
        in_specs=[
            pl.BlockSpec((_BM, _L), lambda i, j: (i, 0)),
            pl.BlockSpec((_BM, _L), lambda i, j: (i, 0)),
            pl.BlockSpec((1, _BN), lambda i, j: (0, j)),
            pl.BlockSpec((1, _BN), lambda i, j: (0, j)),
        ],
        out_specs=pl.BlockSpec((1, 8, _L), lambda i, j: (i, 0, 0)),
        out_shape=jax.ShapeDtypeStruct((ni, 8, _L), jnp.float32),
        scratch_shapes=[
            pltpu.VMEM((_BM, _L), jnp.float32),
            pltpu.VMEM((_BM, _L), jnp.float32),
            pltpu.VMEM((_BM, _L), jnp.float32),
        ],
        compiler_params=pltpu.CompilerParams(
            dimension_semantics=("parallel", "arbitrary")),
    )(xbc, ybc, xrow, yrow)
    total = jnp.sum(out)
    denom = e * (e - 1) / 2
    return total * 0.5 / denom
